# baseline (speedup 1.0000x reference)
import functools

import jax
import jax.numpy as jnp
from jax import lax
from jax.experimental import pallas as pl
from jax.experimental.pallas import tpu as pltpu
from jax.experimental.pallas import tpu_sc as plsc

N = 100000
E = 6400000
B = 4096

NC = 2
NS = 16
NW = NC * NS

GW = 128
CHUNK_E = 2048
TOTAL_CHUNKS = E // CHUNK_E
FB = 4
NL = 8
LBUF = CHUNK_E // NL + GW
ACC_STRIPE = 6272
ACC_PAD = ACC_STRIPE * NS
DEAD = N

_mesh = plsc.VectorSubcoreMesh(core_axis_name="c", subcore_axis_name="s")


@functools.partial(
    pl.kernel,
    out_type=[
        jax.ShapeDtypeStruct((NC, B), jnp.float32),
        jax.ShapeDtypeStruct((NC, B), jnp.float32),
        jax.ShapeDtypeStruct((B,), jnp.float32),
        jax.ShapeDtypeStruct((B,), jnp.float32),
    ],
    mesh=_mesh,
    compiler_params=pltpu.CompilerParams(needs_layout_passes=False),
    scratch_types=[
        pltpu.VMEM((N,), jnp.int32),
        pltpu.VMEM((3 * CHUNK_E,), jnp.int32),
        pltpu.VMEM((3 * CHUNK_E,), jnp.int32),
        pltpu.VMEM((NL * LBUF,), jnp.int32),
        pltpu.VMEM((NL * LBUF,), jnp.int32),
        pltpu.VMEM((FB, GW), jnp.int32),
        pltpu.VMEM((FB, GW), jnp.int32),
        pltpu.VMEM((FB, GW), jnp.float32),
        pltpu.VMEM((128,), jnp.int32),
        pltpu.VMEM((128,), jnp.float32),
        pltpu.VMEM((128,), jnp.float32),
        pltpu.VMEM((128,), jnp.float32),
        pltpu.VMEM((128,), jnp.float32),
        pltpu.VMEM_SHARED((ACC_PAD,), jnp.float32),
        pltpu.SemaphoreType.DMA((3,)),
        pltpu.SemaphoreType.DMA((FB,)),
        pltpu.SemaphoreType.DMA((FB,)),
    ],
)
def _sc_pagerank(e2, x_hbm, nw_hbm, home_hbm, away_hbm, zeros_f, zeros_i,
                 out_acc_h, out_acc_a, out_xnw_h, out_xnw_a,
                 flag_v, rows_v, cols_v, rbuf, cbuf, rfire, cfire, vfire,
                 idx_v, nw_v, g_v, xg_v, xnw_v, acc_sh,
                 sem_in, sem_g, sem_s):
    cid = lax.axis_index("c")
    sid = lax.axis_index("s")
    wid = cid * NS + sid

    pltpu.sync_copy(zeros_i, flag_v)
    pltpu.sync_copy(home_hbm, rows_v.at[pl.ds(0, B)])
    pltpu.sync_copy(away_hbm, cols_v.at[pl.ds(0, B)])
    ones16 = jnp.ones((16,), jnp.int32)

    def set_body(i, _):
        plsc.store_scatter(flag_v, [rows_v[pl.ds(16 * i, 16)]], ones16)
        plsc.store_scatter(flag_v, [cols_v[pl.ds(16 * i, 16)]], ones16)
        return 0

    lax.fori_loop(0, B // 16, set_body, 0)

    z0 = sid * ACC_STRIPE
    pltpu.sync_copy(zeros_f.at[pl.ds(z0, ACC_STRIPE)],
                    acc_sh.at[pl.ds(z0, ACC_STRIPE)])
    plsc.subcore_barrier()

    n_chunks = 97 + jnp.where(wid < TOTAL_CHUNKS - 97 * NW, 1, 0)

    def load_chunk(c, slot):
        r0 = (c * NW + wid) * CHUNK_E
        b0 = slot * CHUNK_E
        pltpu.async_copy(e2.at[0, pl.ds(r0, CHUNK_E)],
                         rows_v.at[pl.ds(b0, CHUNK_E)], sem_in.at[slot])
        pltpu.async_copy(e2.at[1, pl.ds(r0, CHUNK_E)],
                         cols_v.at[pl.ds(b0, CHUNK_E)], sem_in.at[slot])

    def wait_chunk(c, slot):
        r0 = (c * NW + wid) * CHUNK_E
        b0 = slot * CHUNK_E
        pltpu.make_async_copy(e2.at[0, pl.ds(r0, CHUNK_E)],
                              rows_v.at[pl.ds(b0, CHUNK_E)],
                              sem_in.at[slot]).wait()
        pltpu.make_async_copy(e2.at[1, pl.ds(r0, CHUNK_E)],
                              cols_v.at[pl.ds(b0, CHUNK_E)],
                              sem_in.at[slot]).wait()

    def gather_wait(slot):
        pltpu.make_async_copy(x_hbm.at[rfire.at[slot]], vfire.at[slot],
                              sem_g.at[slot]).wait()

    def scatter_wait(slot):
        pltpu.make_async_copy(vfire.at[slot], acc_sh.at[cfire.at[slot]],
                              sem_s.at[slot]).wait()

    def chain_scatter(t):
        @pl.when(t >= 1)
        def _():
            pslot = lax.rem(t - 1, FB)
            gather_wait(pslot)
            pltpu.async_copy(vfire.at[pslot], acc_sh.at[cfire.at[pslot]],
                             sem_s.at[pslot], add=True)

    def flush_lane(ln, pcnt, t):
        nfull = pcnt // GW

        def fbody(j, t):
            slot = lax.rem(t, FB)

            @pl.when(t >= FB)
            def _():
                scatter_wait(slot)
            chain_scatter(t)
            for k in range(GW // 16):
                rfire.at[slot][pl.ds(16 * k, 16)] = (
                    rbuf[pl.ds(ln * LBUF + j * GW + 16 * k, 16)])
                cfire.at[slot][pl.ds(16 * k, 16)] = (
                    cbuf[pl.ds(ln * LBUF + j * GW + 16 * k, 16)])
            pltpu.async_copy(x_hbm.at[rfire.at[slot]], vfire.at[slot],
                             sem_g.at[slot])
            return t + 1

        t = lax.fori_loop(0, nfull, fbody, t)
        rem = pcnt - nfull * GW

        @pl.when(nfull > 0)
        def _():
            def mv(k, _):
                rbuf[pl.ds(ln * LBUF + 16 * k, 16)] = (
                    rbuf[pl.ds(ln * LBUF + nfull * GW + 16 * k, 16)])
                cbuf[pl.ds(ln * LBUF + 16 * k, 16)] = (
                    cbuf[pl.ds(ln * LBUF + nfull * GW + 16 * k, 16)])
                return 0
            lax.fori_loop(0, (rem + 15) // 16, mv, 0)

        return rem, t

    load_chunk(0, 0)

    def chunk_body(c, carry):
        cnts, t = carry
        m = lax.rem(c, 3)
        wait_chunk(c, m)

        @pl.when(c + 1 < n_chunks)
        def _():
            load_chunk(c + 1, lax.rem(c + 1, 3))

        b0 = m * CHUNK_E

        def scan_body(i, cnts):
            out = []
            for ln in range(NL):
                o = b0 + 16 * (i * NL + ln)
                row = rows_v[pl.ds(o, 16)]
                col = cols_v[pl.ds(o, 16)]
                fl = plsc.load_gather(flag_v, [col])
                msk = fl > 0
                plsc.store_compressed(rbuf.at[pl.ds(ln * LBUF + cnts[ln], 16)],
                                      row, mask=msk)
                plsc.store_compressed(cbuf.at[pl.ds(ln * LBUF + cnts[ln], 16)],
                                      col, mask=msk)
                pcv = plsc.all_reduce_population_count(msk)
                out.append(cnts[ln] + pcv[0])
            return tuple(out)

        cnts = lax.fori_loop(0, CHUNK_E // 16 // NL, scan_body, cnts)
        new_cnts = []
        for ln in range(NL):
            cnt, t = flush_lane(ln, cnts[ln], t)
            new_cnts.append(cnt)
        return (tuple(new_cnts), t)

    cnts, t = lax.fori_loop(
        0, n_chunks, chunk_body,
        (tuple(jnp.int32(0) for _ in range(NL)), jnp.int32(0)))

    lane = lax.iota(jnp.int32, 16)
    for ln in range(NL):
        pcnt = cnts[ln]
        rem16 = lax.rem(pcnt, 16)
        a0 = pcnt - rem16
        keep = lane < rem16
        rbuf[pl.ds(ln * LBUF + a0, 16)] = jnp.where(
            keep, rbuf[pl.ds(ln * LBUF + a0, 16)], 0)
        cbuf[pl.ds(ln * LBUF + a0, 16)] = jnp.where(
            keep, cbuf[pl.ds(ln * LBUF + a0, 16)], DEAD)
        for k in range(1, GW // 16 + 1):
            rbuf[pl.ds(ln * LBUF + a0 + 16 * k, 16)] = (
                jnp.zeros((16,), jnp.int32))
            cbuf[pl.ds(ln * LBUF + a0 + 16 * k, 16)] = jnp.full(
                (16,), DEAD, jnp.int32)
        _, t = flush_lane(ln, jnp.where(pcnt > 0, GW, 0), t)

    chain_scatter(t)
    for s in range(FB):
        @pl.when(s < t)
        def _():
            scatter_wait(jnp.int32(s))
    plsc.subcore_barrier()

    for role_hbm, out_acc, out_xnw in (
        (home_hbm, out_acc_h, out_xnw_h),
        (away_hbm, out_acc_a, out_xnw_a),
    ):
        for q in range(2):
            base = sid * 256 + q * 128
            pltpu.sync_copy(role_hbm.at[pl.ds(base, 128)], idx_v)
            pltpu.sync_copy(acc_sh.at[idx_v], g_v)
            pltpu.sync_copy(g_v, out_acc.at[cid, pl.ds(base, 128)])

            @pl.when(cid == 0)
            def _():
                pltpu.sync_copy(nw_hbm.at[idx_v], nw_v)
                pltpu.sync_copy(x_hbm.at[idx_v], xg_v)
                for k in range(8):
                    xnw_v[pl.ds(16 * k, 16)] = (
                        xg_v[pl.ds(16 * k, 16)] * nw_v[pl.ds(16 * k, 16)])
                pltpu.sync_copy(xnw_v, out_xnw.at[pl.ds(base, 128)])


def kernel(x, node_weight, edge_index, home, away, W, b):
    x_flat = x.reshape(N)
    nw_flat = node_weight.reshape(N)
    zeros_f = jnp.zeros((ACC_PAD,), jnp.float32)
    zeros_i = jnp.zeros((N,), jnp.int32)

    acc_h, acc_a, xnw_h, xnw_a = _sc_pagerank(
        edge_index, x_flat, nw_flat, home, away, zeros_f, zeros_i)

    w00 = W[0, 0]
    th = xnw_h + (acc_h[0] + acc_h[1]) * w00 + b[0]
    ta = xnw_a + (acc_a[0] + acc_a[1]) * w00 + b[0]
    out = jnp.sign(jax.nn.sigmoid(th) - jax.nn.sigmoid(ta))
    return out.reshape(B, 1)


# speedup vs baseline: 2.1990x; 2.1990x over previous
import functools

import jax
import jax.numpy as jnp
from jax import lax
from jax.experimental import pallas as pl
from jax.experimental.pallas import tpu as pltpu
from jax.experimental.pallas import tpu_sc as plsc

N = 100000
E = 6400000
B = 4096

NC = 2
NS = 16
NW = NC * NS

GW = 128
CHUNK_E = 2048
TOTAL_CHUNKS = E // CHUNK_E
FB = 4
PBUF = CHUNK_E + GW
ACC_STRIPE = 6272
ACC_PAD = ACC_STRIPE * NS
DEAD = N

_mesh = plsc.VectorSubcoreMesh(core_axis_name="c", subcore_axis_name="s")


@functools.partial(
    pl.kernel,
    out_type=[
        jax.ShapeDtypeStruct((NC, B), jnp.float32),
        jax.ShapeDtypeStruct((NC, B), jnp.float32),
        jax.ShapeDtypeStruct((B,), jnp.float32),
        jax.ShapeDtypeStruct((B,), jnp.float32),
    ],
    mesh=_mesh,
    compiler_params=pltpu.CompilerParams(needs_layout_passes=False),
    scratch_types=[
        pltpu.VMEM((N,), jnp.int32),
        pltpu.VMEM((3 * CHUNK_E,), jnp.int32),
        pltpu.VMEM((3 * CHUNK_E,), jnp.int32),
        pltpu.VMEM((PBUF,), jnp.int32),
        pltpu.VMEM((PBUF,), jnp.int32),
        pltpu.VMEM((FB, GW), jnp.int32),
        pltpu.VMEM((FB, GW), jnp.int32),
        pltpu.VMEM((FB, GW), jnp.float32),
        pltpu.VMEM((128,), jnp.int32),
        pltpu.VMEM((128,), jnp.float32),
        pltpu.VMEM((128,), jnp.float32),
        pltpu.VMEM((128,), jnp.float32),
        pltpu.VMEM((128,), jnp.float32),
        pltpu.VMEM_SHARED((ACC_PAD,), jnp.float32),
        pltpu.SemaphoreType.DMA((3,)),
        pltpu.SemaphoreType.DMA((FB,)),
        pltpu.SemaphoreType.DMA((FB,)),
    ],
)
def _sc_pagerank(e2, x_hbm, nw_hbm, home_hbm, away_hbm, zeros_f, zeros_i,
                 out_acc_h, out_acc_a, out_xnw_h, out_xnw_a,
                 flag_v, rows_v, cols_v, rbuf, cbuf, rfire, cfire, vfire,
                 idx_v, nw_v, g_v, xg_v, xnw_v, acc_sh,
                 sem_in, sem_g, sem_s):
    cid = lax.axis_index("c")
    sid = lax.axis_index("s")
    wid = cid * NS + sid

    pltpu.sync_copy(zeros_i, flag_v)
    pltpu.sync_copy(home_hbm, rows_v.at[pl.ds(0, B)])
    pltpu.sync_copy(away_hbm, cols_v.at[pl.ds(0, B)])
    ones16 = jnp.ones((16,), jnp.int32)

    def set_body(i, _):
        plsc.store_scatter(flag_v, [rows_v[pl.ds(16 * i, 16)]], ones16)
        plsc.store_scatter(flag_v, [cols_v[pl.ds(16 * i, 16)]], ones16)
        return 0

    lax.fori_loop(0, B // 16, set_body, 0)

    z0 = sid * ACC_STRIPE
    pltpu.sync_copy(zeros_f.at[pl.ds(z0, ACC_STRIPE)],
                    acc_sh.at[pl.ds(z0, ACC_STRIPE)])
    plsc.subcore_barrier()

    n_chunks = 97 + jnp.where(wid < TOTAL_CHUNKS - 97 * NW, 1, 0)

    def load_chunk(c, slot):
        r0 = (c * NW + wid) * CHUNK_E
        b0 = slot * CHUNK_E
        pltpu.async_copy(e2.at[0, pl.ds(r0, CHUNK_E)],
                         rows_v.at[pl.ds(b0, CHUNK_E)], sem_in.at[slot])
        pltpu.async_copy(e2.at[1, pl.ds(r0, CHUNK_E)],
                         cols_v.at[pl.ds(b0, CHUNK_E)], sem_in.at[slot])

    def wait_chunk(c, slot):
        r0 = (c * NW + wid) * CHUNK_E
        b0 = slot * CHUNK_E
        pltpu.make_async_copy(e2.at[0, pl.ds(r0, CHUNK_E)],
                              rows_v.at[pl.ds(b0, CHUNK_E)],
                              sem_in.at[slot]).wait()
        pltpu.make_async_copy(e2.at[1, pl.ds(r0, CHUNK_E)],
                              cols_v.at[pl.ds(b0, CHUNK_E)],
                              sem_in.at[slot]).wait()

    def gather_wait(slot):
        pltpu.make_async_copy(x_hbm.at[rfire.at[slot]], vfire.at[slot],
                              sem_g.at[slot]).wait()

    def scatter_wait(slot):
        pltpu.make_async_copy(vfire.at[slot], acc_sh.at[cfire.at[slot]],
                              sem_s.at[slot]).wait()

    def chain_scatter(t):
        @pl.when(t >= 1)
        def _():
            pslot = lax.rem(t - 1, FB)
            gather_wait(pslot)
            pltpu.async_copy(vfire.at[pslot], acc_sh.at[cfire.at[pslot]],
                             sem_s.at[pslot], add=True)

    def flush(pcnt, t):
        nfull = pcnt // GW

        def fbody(j, t):
            slot = lax.rem(t, FB)

            @pl.when(t >= FB)
            def _():
                scatter_wait(slot)
            chain_scatter(t)
            for k in range(GW // 16):
                rfire.at[slot][pl.ds(16 * k, 16)] = (
                    rbuf[pl.ds(j * GW + 16 * k, 16)])
                cfire.at[slot][pl.ds(16 * k, 16)] = (
                    cbuf[pl.ds(j * GW + 16 * k, 16)])
            pltpu.async_copy(x_hbm.at[rfire.at[slot]], vfire.at[slot],
                             sem_g.at[slot])
            return t + 1

        t = lax.fori_loop(0, nfull, fbody, t)
        rem = pcnt - nfull * GW

        @pl.when(nfull > 0)
        def _():
            def mv(k, _):
                rbuf[pl.ds(16 * k, 16)] = rbuf[pl.ds(nfull * GW + 16 * k, 16)]
                cbuf[pl.ds(16 * k, 16)] = cbuf[pl.ds(nfull * GW + 16 * k, 16)]
                return 0
            lax.fori_loop(0, (rem + 15) // 16, mv, 0)

        return rem, t

    load_chunk(0, 0)

    def chunk_body(c, carry):
        pcnt, t = carry
        m = lax.rem(c, 3)
        wait_chunk(c, m)

        @pl.when(c + 1 < n_chunks)
        def _():
            load_chunk(c + 1, lax.rem(c + 1, 3))

        b0 = m * CHUNK_E

        @plsc.parallel_loop(0, CHUNK_E // 16, unroll=8, carry=pcnt)
        def pcnt(i, pcnt):
            o = b0 + 16 * i
            row = rows_v[pl.ds(o, 16)]
            col = cols_v[pl.ds(o, 16)]
            fl = plsc.load_gather(flag_v, [col])
            msk = fl > 0
            plsc.store_compressed(rbuf.at[pl.ds(pcnt, 16)], row, mask=msk)
            plsc.store_compressed(cbuf.at[pl.ds(pcnt, 16)], col, mask=msk)
            pcv = plsc.all_reduce_population_count(msk)
            return pcnt + pcv[0]

        pcnt, t = flush(pcnt, t)
        return (pcnt, t)

    pcnt, t = lax.fori_loop(0, n_chunks, chunk_body,
                            (jnp.int32(0), jnp.int32(0)))

    lane = lax.iota(jnp.int32, 16)
    rem16 = lax.rem(pcnt, 16)
    a0 = pcnt - rem16
    keep = lane < rem16
    rbuf[pl.ds(a0, 16)] = jnp.where(keep, rbuf[pl.ds(a0, 16)], 0)
    cbuf[pl.ds(a0, 16)] = jnp.where(keep, cbuf[pl.ds(a0, 16)], DEAD)
    for k in range(1, GW // 16 + 1):
        rbuf[pl.ds(a0 + 16 * k, 16)] = jnp.zeros((16,), jnp.int32)
        cbuf[pl.ds(a0 + 16 * k, 16)] = jnp.full((16,), DEAD, jnp.int32)
    _, t = flush(jnp.where(pcnt > 0, GW, 0), t)

    chain_scatter(t)
    for s in range(FB):
        @pl.when(s < t)
        def _():
            scatter_wait(jnp.int32(s))
    plsc.subcore_barrier()

    for role_hbm, out_acc, out_xnw in (
        (home_hbm, out_acc_h, out_xnw_h),
        (away_hbm, out_acc_a, out_xnw_a),
    ):
        for q in range(2):
            base = sid * 256 + q * 128
            pltpu.sync_copy(role_hbm.at[pl.ds(base, 128)], idx_v)
            pltpu.sync_copy(acc_sh.at[idx_v], g_v)
            pltpu.sync_copy(g_v, out_acc.at[cid, pl.ds(base, 128)])

            @pl.when(cid == 0)
            def _():
                pltpu.sync_copy(nw_hbm.at[idx_v], nw_v)
                pltpu.sync_copy(x_hbm.at[idx_v], xg_v)
                for k in range(8):
                    xnw_v[pl.ds(16 * k, 16)] = (
                        xg_v[pl.ds(16 * k, 16)] * nw_v[pl.ds(16 * k, 16)])
                pltpu.sync_copy(xnw_v, out_xnw.at[pl.ds(base, 128)])


def kernel(x, node_weight, edge_index, home, away, W, b):
    x_flat = x.reshape(N)
    nw_flat = node_weight.reshape(N)
    zeros_f = jnp.zeros((ACC_PAD,), jnp.float32)
    zeros_i = jnp.zeros((N,), jnp.int32)

    acc_h, acc_a, xnw_h, xnw_a = _sc_pagerank(
        edge_index, x_flat, nw_flat, home, away, zeros_f, zeros_i)

    w00 = W[0, 0]
    th = xnw_h + (acc_h[0] + acc_h[1]) * w00 + b[0]
    ta = xnw_a + (acc_a[0] + acc_a[1]) * w00 + b[0]
    out = jnp.sign(jax.nn.sigmoid(th) - jax.nn.sigmoid(ta))
    return out.reshape(B, 1)


# speedup vs baseline: 2.2001x; 1.0005x over previous
import functools

import jax
import jax.numpy as jnp
from jax import lax
from jax.experimental import pallas as pl
from jax.experimental.pallas import tpu as pltpu
from jax.experimental.pallas import tpu_sc as plsc

N = 100000
E = 6400000
B = 4096

NC = 2
NS = 16
NW = NC * NS

GW = 128
CHUNK_E = 2048
TOTAL_CHUNKS = E // CHUNK_E
FB = 4
PBUF = CHUNK_E + GW
ACC_STRIPE = 6272
ACC_PAD = ACC_STRIPE * NS
DEAD = N

_mesh = plsc.VectorSubcoreMesh(core_axis_name="c", subcore_axis_name="s")


@functools.partial(
    pl.kernel,
    out_type=[
        jax.ShapeDtypeStruct((NC, B), jnp.float32),
        jax.ShapeDtypeStruct((NC, B), jnp.float32),
        jax.ShapeDtypeStruct((B,), jnp.float32),
        jax.ShapeDtypeStruct((B,), jnp.float32),
    ],
    mesh=_mesh,
    compiler_params=pltpu.CompilerParams(needs_layout_passes=False),
    scratch_types=[
        pltpu.VMEM((N,), jnp.int32),
        pltpu.VMEM((3 * CHUNK_E,), jnp.int32),
        pltpu.VMEM((3 * CHUNK_E,), jnp.int32),
        pltpu.VMEM((PBUF,), jnp.int32),
        pltpu.VMEM((PBUF,), jnp.int32),
        pltpu.VMEM((FB, GW), jnp.int32),
        pltpu.VMEM((FB, GW), jnp.int32),
        pltpu.VMEM((FB, GW), jnp.float32),
        pltpu.VMEM((128,), jnp.int32),
        pltpu.VMEM((128,), jnp.float32),
        pltpu.VMEM((128,), jnp.float32),
        pltpu.VMEM((128,), jnp.float32),
        pltpu.VMEM((128,), jnp.float32),
        pltpu.VMEM_SHARED((ACC_PAD,), jnp.float32),
        pltpu.SemaphoreType.DMA((3,)),
        pltpu.SemaphoreType.DMA((FB,)),
        pltpu.SemaphoreType.DMA((FB,)),
    ],
)
def _sc_pagerank(e2, x_hbm, nw_hbm, home_hbm, away_hbm, zeros_f, zeros_i,
                 out_acc_h, out_acc_a, out_xnw_h, out_xnw_a,
                 flag_v, rows_v, cols_v, rbuf, cbuf, rfire, cfire, vfire,
                 idx_v, nw_v, g_v, xg_v, xnw_v, acc_sh,
                 sem_in, sem_g, sem_s):
    cid = lax.axis_index("c")
    sid = lax.axis_index("s")
    wid = cid * NS + sid

    pltpu.sync_copy(zeros_i, flag_v)
    pltpu.sync_copy(home_hbm, rows_v.at[pl.ds(0, B)])
    pltpu.sync_copy(away_hbm, cols_v.at[pl.ds(0, B)])
    ones16 = jnp.ones((16,), jnp.int32)

    def set_body(i, _):
        plsc.store_scatter(flag_v, [rows_v[pl.ds(16 * i, 16)]], ones16)
        plsc.store_scatter(flag_v, [cols_v[pl.ds(16 * i, 16)]], ones16)
        return 0

    lax.fori_loop(0, B // 16, set_body, 0)

    z0 = sid * ACC_STRIPE
    pltpu.sync_copy(zeros_f.at[pl.ds(z0, ACC_STRIPE)],
                    acc_sh.at[pl.ds(z0, ACC_STRIPE)])
    plsc.subcore_barrier()

    n_chunks = 97 + jnp.where(wid < TOTAL_CHUNKS - 97 * NW, 1, 0)

    def load_chunk(c, slot):
        r0 = (c * NW + wid) * CHUNK_E
        b0 = slot * CHUNK_E
        pltpu.async_copy(e2.at[0, pl.ds(r0, CHUNK_E)],
                         rows_v.at[pl.ds(b0, CHUNK_E)], sem_in.at[slot])
        pltpu.async_copy(e2.at[1, pl.ds(r0, CHUNK_E)],
                         cols_v.at[pl.ds(b0, CHUNK_E)], sem_in.at[slot])

    def wait_chunk(c, slot):
        r0 = (c * NW + wid) * CHUNK_E
        b0 = slot * CHUNK_E
        pltpu.make_async_copy(e2.at[0, pl.ds(r0, CHUNK_E)],
                              rows_v.at[pl.ds(b0, CHUNK_E)],
                              sem_in.at[slot]).wait()
        pltpu.make_async_copy(e2.at[1, pl.ds(r0, CHUNK_E)],
                              cols_v.at[pl.ds(b0, CHUNK_E)],
                              sem_in.at[slot]).wait()

    def gather_wait(slot):
        pltpu.make_async_copy(x_hbm.at[rfire.at[slot]], vfire.at[slot],
                              sem_g.at[slot]).wait()

    def scatter_wait(slot):
        pltpu.make_async_copy(vfire.at[slot], acc_sh.at[cfire.at[slot]],
                              sem_s.at[slot]).wait()

    def chain_scatter(t):
        @pl.when(t >= 1)
        def _():
            pslot = lax.rem(t - 1, FB)
            gather_wait(pslot)
            pltpu.async_copy(vfire.at[pslot], acc_sh.at[cfire.at[pslot]],
                             sem_s.at[pslot], add=True)

    def flush(pcnt, t):
        nfull = pcnt // GW

        def fbody(j, t):
            slot = lax.rem(t, FB)

            @pl.when(t >= FB)
            def _():
                scatter_wait(slot)
            chain_scatter(t)
            for k in range(GW // 16):
                rfire.at[slot][pl.ds(16 * k, 16)] = (
                    rbuf[pl.ds(j * GW + 16 * k, 16)])
                cfire.at[slot][pl.ds(16 * k, 16)] = (
                    cbuf[pl.ds(j * GW + 16 * k, 16)])
            pltpu.async_copy(x_hbm.at[rfire.at[slot]], vfire.at[slot],
                             sem_g.at[slot])
            return t + 1

        t = lax.fori_loop(0, nfull, fbody, t)
        rem = pcnt - nfull * GW

        @pl.when(nfull > 0)
        def _():
            def mv(k, _):
                rbuf[pl.ds(16 * k, 16)] = rbuf[pl.ds(nfull * GW + 16 * k, 16)]
                cbuf[pl.ds(16 * k, 16)] = cbuf[pl.ds(nfull * GW + 16 * k, 16)]
                return 0
            lax.fori_loop(0, (rem + 15) // 16, mv, 0)

        return rem, t

    load_chunk(0, 0)

    def chunk_body(c, carry):
        pcnt, t = carry
        m = lax.rem(c, 3)
        wait_chunk(c, m)

        @pl.when(c + 1 < n_chunks)
        def _():
            load_chunk(c + 1, lax.rem(c + 1, 3))

        b0 = m * CHUNK_E

        @plsc.parallel_loop(0, CHUNK_E // 16, unroll=16, carry=pcnt)
        def pcnt(i, pcnt):
            o = b0 + 16 * i
            row = rows_v[pl.ds(o, 16)]
            col = cols_v[pl.ds(o, 16)]
            fl = plsc.load_gather(flag_v, [col])
            msk = fl > 0
            plsc.store_compressed(rbuf.at[pl.ds(pcnt, 16)], row, mask=msk)
            plsc.store_compressed(cbuf.at[pl.ds(pcnt, 16)], col, mask=msk)
            pcv = plsc.all_reduce_population_count(msk)
            return pcnt + pcv[0]

        pcnt, t = flush(pcnt, t)
        return (pcnt, t)

    pcnt, t = lax.fori_loop(0, n_chunks, chunk_body,
                            (jnp.int32(0), jnp.int32(0)))

    lane = lax.iota(jnp.int32, 16)
    rem16 = lax.rem(pcnt, 16)
    a0 = pcnt - rem16
    keep = lane < rem16
    rbuf[pl.ds(a0, 16)] = jnp.where(keep, rbuf[pl.ds(a0, 16)], 0)
    cbuf[pl.ds(a0, 16)] = jnp.where(keep, cbuf[pl.ds(a0, 16)], DEAD)
    for k in range(1, GW // 16 + 1):
        rbuf[pl.ds(a0 + 16 * k, 16)] = jnp.zeros((16,), jnp.int32)
        cbuf[pl.ds(a0 + 16 * k, 16)] = jnp.full((16,), DEAD, jnp.int32)
    _, t = flush(jnp.where(pcnt > 0, GW, 0), t)

    chain_scatter(t)
    for s in range(FB):
        @pl.when(s < t)
        def _():
            scatter_wait(jnp.int32(s))
    plsc.subcore_barrier()

    for role_hbm, out_acc, out_xnw in (
        (home_hbm, out_acc_h, out_xnw_h),
        (away_hbm, out_acc_a, out_xnw_a),
    ):
        for q in range(2):
            base = sid * 256 + q * 128
            pltpu.sync_copy(role_hbm.at[pl.ds(base, 128)], idx_v)
            pltpu.sync_copy(acc_sh.at[idx_v], g_v)
            pltpu.sync_copy(g_v, out_acc.at[cid, pl.ds(base, 128)])

            @pl.when(cid == 0)
            def _():
                pltpu.sync_copy(nw_hbm.at[idx_v], nw_v)
                pltpu.sync_copy(x_hbm.at[idx_v], xg_v)
                for k in range(8):
                    xnw_v[pl.ds(16 * k, 16)] = (
                        xg_v[pl.ds(16 * k, 16)] * nw_v[pl.ds(16 * k, 16)])
                pltpu.sync_copy(xnw_v, out_xnw.at[pl.ds(base, 128)])


def kernel(x, node_weight, edge_index, home, away, W, b):
    x_flat = x.reshape(N)
    nw_flat = node_weight.reshape(N)
    zeros_f = jnp.zeros((ACC_PAD,), jnp.float32)
    zeros_i = jnp.zeros((N,), jnp.int32)

    acc_h, acc_a, xnw_h, xnw_a = _sc_pagerank(
        edge_index, x_flat, nw_flat, home, away, zeros_f, zeros_i)

    w00 = W[0, 0]
    th = xnw_h + (acc_h[0] + acc_h[1]) * w00 + b[0]
    ta = xnw_a + (acc_a[0] + acc_a[1]) * w00 + b[0]
    out = jnp.sign(jax.nn.sigmoid(th) - jax.nn.sigmoid(ta))
    return out.reshape(B, 1)


# speedup vs baseline: 2.6035x; 1.1833x over previous
import functools

import jax
import jax.numpy as jnp
from jax import lax
from jax.experimental import pallas as pl
from jax.experimental.pallas import tpu as pltpu
from jax.experimental.pallas import tpu_sc as plsc

N = 100000
E = 6400000
B = 4096

NC = 2
NS = 16
NW = NC * NS

GW = 128
CHUNK_E = 2048
TOTAL_CHUNKS = E // CHUNK_E
FB = 8
LAG = 3
PBUF = CHUNK_E + GW
ACC_STRIPE = 6272
ACC_PAD = ACC_STRIPE * NS
DEAD = N

_mesh = plsc.VectorSubcoreMesh(core_axis_name="c", subcore_axis_name="s")


@functools.partial(
    pl.kernel,
    out_type=[
        jax.ShapeDtypeStruct((NC, B), jnp.float32),
        jax.ShapeDtypeStruct((NC, B), jnp.float32),
        jax.ShapeDtypeStruct((B,), jnp.float32),
        jax.ShapeDtypeStruct((B,), jnp.float32),
    ],
    mesh=_mesh,
    compiler_params=pltpu.CompilerParams(needs_layout_passes=False),
    scratch_types=[
        pltpu.VMEM((N,), jnp.int32),
        pltpu.VMEM((3 * CHUNK_E,), jnp.int32),
        pltpu.VMEM((3 * CHUNK_E,), jnp.int32),
        pltpu.VMEM((PBUF,), jnp.int32),
        pltpu.VMEM((PBUF,), jnp.int32),
        pltpu.VMEM((FB, GW), jnp.int32),
        pltpu.VMEM((FB, GW), jnp.int32),
        pltpu.VMEM((FB, GW), jnp.float32),
        pltpu.VMEM((128,), jnp.int32),
        pltpu.VMEM((128,), jnp.float32),
        pltpu.VMEM((128,), jnp.float32),
        pltpu.VMEM((128,), jnp.float32),
        pltpu.VMEM((128,), jnp.float32),
        pltpu.VMEM_SHARED((ACC_PAD,), jnp.float32),
        pltpu.SemaphoreType.DMA((3,)),
        pltpu.SemaphoreType.DMA((FB,)),
        pltpu.SemaphoreType.DMA((FB,)),
    ],
)
def _sc_pagerank(e2, x_hbm, nw_hbm, home_hbm, away_hbm, zeros_f, zeros_i,
                 out_acc_h, out_acc_a, out_xnw_h, out_xnw_a,
                 flag_v, rows_v, cols_v, rbuf, cbuf, rfire, cfire, vfire,
                 idx_v, nw_v, g_v, xg_v, xnw_v, acc_sh,
                 sem_in, sem_g, sem_s):
    cid = lax.axis_index("c")
    sid = lax.axis_index("s")
    wid = cid * NS + sid

    pltpu.sync_copy(zeros_i, flag_v)
    pltpu.sync_copy(home_hbm, rows_v.at[pl.ds(0, B)])
    pltpu.sync_copy(away_hbm, cols_v.at[pl.ds(0, B)])
    ones16 = jnp.ones((16,), jnp.int32)

    def set_body(i, _):
        plsc.store_scatter(flag_v, [rows_v[pl.ds(16 * i, 16)]], ones16)
        plsc.store_scatter(flag_v, [cols_v[pl.ds(16 * i, 16)]], ones16)
        return 0

    lax.fori_loop(0, B // 16, set_body, 0)

    z0 = sid * ACC_STRIPE
    pltpu.sync_copy(zeros_f.at[pl.ds(z0, ACC_STRIPE)],
                    acc_sh.at[pl.ds(z0, ACC_STRIPE)])
    plsc.subcore_barrier()

    n_chunks = 97 + jnp.where(wid < TOTAL_CHUNKS - 97 * NW, 1, 0)

    def load_chunk(c, slot):
        r0 = (c * NW + wid) * CHUNK_E
        b0 = slot * CHUNK_E
        pltpu.async_copy(e2.at[0, pl.ds(r0, CHUNK_E)],
                         rows_v.at[pl.ds(b0, CHUNK_E)], sem_in.at[slot])
        pltpu.async_copy(e2.at[1, pl.ds(r0, CHUNK_E)],
                         cols_v.at[pl.ds(b0, CHUNK_E)], sem_in.at[slot])

    def wait_chunk(c, slot):
        r0 = (c * NW + wid) * CHUNK_E
        b0 = slot * CHUNK_E
        pltpu.make_async_copy(e2.at[0, pl.ds(r0, CHUNK_E)],
                              rows_v.at[pl.ds(b0, CHUNK_E)],
                              sem_in.at[slot]).wait()
        pltpu.make_async_copy(e2.at[1, pl.ds(r0, CHUNK_E)],
                              cols_v.at[pl.ds(b0, CHUNK_E)],
                              sem_in.at[slot]).wait()

    def gather_wait(slot):
        pltpu.make_async_copy(x_hbm.at[rfire.at[slot]], vfire.at[slot],
                              sem_g.at[slot]).wait()

    def scatter_wait(slot):
        pltpu.make_async_copy(vfire.at[slot], acc_sh.at[cfire.at[slot]],
                              sem_s.at[slot]).wait()

    def chain_scatter(t):
        @pl.when(t >= LAG)
        def _():
            pslot = lax.rem(t - LAG, FB)
            gather_wait(pslot)
            pltpu.async_copy(vfire.at[pslot], acc_sh.at[cfire.at[pslot]],
                             sem_s.at[pslot], add=True)

    def flush(pcnt, t):
        nfull = pcnt // GW

        def fbody(j, t):
            slot = lax.rem(t, FB)

            @pl.when(t >= FB)
            def _():
                scatter_wait(slot)
            chain_scatter(t)
            for k in range(GW // 16):
                rfire.at[slot][pl.ds(16 * k, 16)] = (
                    rbuf[pl.ds(j * GW + 16 * k, 16)])
                cfire.at[slot][pl.ds(16 * k, 16)] = (
                    cbuf[pl.ds(j * GW + 16 * k, 16)])
            pltpu.async_copy(x_hbm.at[rfire.at[slot]], vfire.at[slot],
                             sem_g.at[slot])
            return t + 1

        t = lax.fori_loop(0, nfull, fbody, t)
        rem = pcnt - nfull * GW

        @pl.when(nfull > 0)
        def _():
            def mv(k, _):
                rbuf[pl.ds(16 * k, 16)] = rbuf[pl.ds(nfull * GW + 16 * k, 16)]
                cbuf[pl.ds(16 * k, 16)] = cbuf[pl.ds(nfull * GW + 16 * k, 16)]
                return 0
            lax.fori_loop(0, (rem + 15) // 16, mv, 0)

        return rem, t

    load_chunk(0, 0)

    def chunk_body(c, carry):
        pcnt, t = carry
        m = lax.rem(c, 3)
        wait_chunk(c, m)

        @pl.when(c + 1 < n_chunks)
        def _():
            load_chunk(c + 1, lax.rem(c + 1, 3))

        b0 = m * CHUNK_E

        @plsc.parallel_loop(0, CHUNK_E // 16, unroll=16, carry=pcnt)
        def pcnt(i, pcnt):
            o = b0 + 16 * i
            row = rows_v[pl.ds(o, 16)]
            col = cols_v[pl.ds(o, 16)]
            fl = plsc.load_gather(flag_v, [col])
            msk = fl > 0
            plsc.store_compressed(rbuf.at[pl.ds(pcnt, 16)], row, mask=msk)
            plsc.store_compressed(cbuf.at[pl.ds(pcnt, 16)], col, mask=msk)
            pcv = plsc.all_reduce_population_count(msk)
            return pcnt + pcv[0]

        pcnt, t = flush(pcnt, t)
        return (pcnt, t)

    pcnt, t = lax.fori_loop(0, n_chunks, chunk_body,
                            (jnp.int32(0), jnp.int32(0)))

    lane = lax.iota(jnp.int32, 16)
    rem16 = lax.rem(pcnt, 16)
    a0 = pcnt - rem16
    keep = lane < rem16
    rbuf[pl.ds(a0, 16)] = jnp.where(keep, rbuf[pl.ds(a0, 16)], 0)
    cbuf[pl.ds(a0, 16)] = jnp.where(keep, cbuf[pl.ds(a0, 16)], DEAD)
    for k in range(1, GW // 16 + 1):
        rbuf[pl.ds(a0 + 16 * k, 16)] = jnp.zeros((16,), jnp.int32)
        cbuf[pl.ds(a0 + 16 * k, 16)] = jnp.full((16,), DEAD, jnp.int32)
    _, t = flush(jnp.where(pcnt > 0, GW, 0), t)

    for d in range(LAG):
        chain_scatter(t + d)
    for s in range(FB):
        @pl.when(s < t)
        def _():
            scatter_wait(jnp.int32(s))
    plsc.subcore_barrier()

    for role_hbm, out_acc, out_xnw in (
        (home_hbm, out_acc_h, out_xnw_h),
        (away_hbm, out_acc_a, out_xnw_a),
    ):
        for q in range(2):
            base = sid * 256 + q * 128
            pltpu.sync_copy(role_hbm.at[pl.ds(base, 128)], idx_v)
            pltpu.sync_copy(acc_sh.at[idx_v], g_v)
            pltpu.sync_copy(g_v, out_acc.at[cid, pl.ds(base, 128)])

            @pl.when(cid == 0)
            def _():
                pltpu.sync_copy(nw_hbm.at[idx_v], nw_v)
                pltpu.sync_copy(x_hbm.at[idx_v], xg_v)
                for k in range(8):
                    xnw_v[pl.ds(16 * k, 16)] = (
                        xg_v[pl.ds(16 * k, 16)] * nw_v[pl.ds(16 * k, 16)])
                pltpu.sync_copy(xnw_v, out_xnw.at[pl.ds(base, 128)])


def kernel(x, node_weight, edge_index, home, away, W, b):
    x_flat = x.reshape(N)
    nw_flat = node_weight.reshape(N)
    zeros_f = jnp.zeros((ACC_PAD,), jnp.float32)
    zeros_i = jnp.zeros((N,), jnp.int32)

    acc_h, acc_a, xnw_h, xnw_a = _sc_pagerank(
        edge_index, x_flat, nw_flat, home, away, zeros_f, zeros_i)

    w00 = W[0, 0]
    th = xnw_h + (acc_h[0] + acc_h[1]) * w00 + b[0]
    ta = xnw_a + (acc_a[0] + acc_a[1]) * w00 + b[0]
    out = jnp.sign(jax.nn.sigmoid(th) - jax.nn.sigmoid(ta))
    return out.reshape(B, 1)


# speedup vs baseline: 2.6036x; 1.0001x over previous
import functools

import jax
import jax.numpy as jnp
from jax import lax
from jax.experimental import pallas as pl
from jax.experimental.pallas import tpu as pltpu
from jax.experimental.pallas import tpu_sc as plsc

N = 100000
E = 6400000
B = 4096

NC = 2
NS = 16
NW = NC * NS

GW = 128
CHUNK_E = 2048
TOTAL_CHUNKS = E // CHUNK_E
FB = 12
LAG = 6
PBUF = CHUNK_E + GW
ACC_STRIPE = 6272
ACC_PAD = ACC_STRIPE * NS
DEAD = N

_mesh = plsc.VectorSubcoreMesh(core_axis_name="c", subcore_axis_name="s")


@functools.partial(
    pl.kernel,
    out_type=[
        jax.ShapeDtypeStruct((NC, B), jnp.float32),
        jax.ShapeDtypeStruct((NC, B), jnp.float32),
        jax.ShapeDtypeStruct((B,), jnp.float32),
        jax.ShapeDtypeStruct((B,), jnp.float32),
    ],
    mesh=_mesh,
    compiler_params=pltpu.CompilerParams(needs_layout_passes=False),
    scratch_types=[
        pltpu.VMEM((N,), jnp.int32),
        pltpu.VMEM((3 * CHUNK_E,), jnp.int32),
        pltpu.VMEM((3 * CHUNK_E,), jnp.int32),
        pltpu.VMEM((PBUF,), jnp.int32),
        pltpu.VMEM((PBUF,), jnp.int32),
        pltpu.VMEM((FB, GW), jnp.int32),
        pltpu.VMEM((FB, GW), jnp.int32),
        pltpu.VMEM((FB, GW), jnp.float32),
        pltpu.VMEM((128,), jnp.int32),
        pltpu.VMEM((128,), jnp.float32),
        pltpu.VMEM((128,), jnp.float32),
        pltpu.VMEM((128,), jnp.float32),
        pltpu.VMEM((128,), jnp.float32),
        pltpu.VMEM_SHARED((ACC_PAD,), jnp.float32),
        pltpu.SemaphoreType.DMA((3,)),
        pltpu.SemaphoreType.DMA((FB,)),
        pltpu.SemaphoreType.DMA((FB,)),
    ],
)
def _sc_pagerank(e2, x_hbm, nw_hbm, home_hbm, away_hbm, zeros_f, zeros_i,
                 out_acc_h, out_acc_a, out_xnw_h, out_xnw_a,
                 flag_v, rows_v, cols_v, rbuf, cbuf, rfire, cfire, vfire,
                 idx_v, nw_v, g_v, xg_v, xnw_v, acc_sh,
                 sem_in, sem_g, sem_s):
    cid = lax.axis_index("c")
    sid = lax.axis_index("s")
    wid = cid * NS + sid

    pltpu.sync_copy(zeros_i, flag_v)
    pltpu.sync_copy(home_hbm, rows_v.at[pl.ds(0, B)])
    pltpu.sync_copy(away_hbm, cols_v.at[pl.ds(0, B)])
    ones16 = jnp.ones((16,), jnp.int32)

    def set_body(i, _):
        plsc.store_scatter(flag_v, [rows_v[pl.ds(16 * i, 16)]], ones16)
        plsc.store_scatter(flag_v, [cols_v[pl.ds(16 * i, 16)]], ones16)
        return 0

    lax.fori_loop(0, B // 16, set_body, 0)

    z0 = sid * ACC_STRIPE
    pltpu.sync_copy(zeros_f.at[pl.ds(z0, ACC_STRIPE)],
                    acc_sh.at[pl.ds(z0, ACC_STRIPE)])
    plsc.subcore_barrier()

    n_chunks = 97 + jnp.where(wid < TOTAL_CHUNKS - 97 * NW, 1, 0)

    def load_chunk(c, slot):
        r0 = (c * NW + wid) * CHUNK_E
        b0 = slot * CHUNK_E
        pltpu.async_copy(e2.at[0, pl.ds(r0, CHUNK_E)],
                         rows_v.at[pl.ds(b0, CHUNK_E)], sem_in.at[slot])
        pltpu.async_copy(e2.at[1, pl.ds(r0, CHUNK_E)],
                         cols_v.at[pl.ds(b0, CHUNK_E)], sem_in.at[slot])

    def wait_chunk(c, slot):
        r0 = (c * NW + wid) * CHUNK_E
        b0 = slot * CHUNK_E
        pltpu.make_async_copy(e2.at[0, pl.ds(r0, CHUNK_E)],
                              rows_v.at[pl.ds(b0, CHUNK_E)],
                              sem_in.at[slot]).wait()
        pltpu.make_async_copy(e2.at[1, pl.ds(r0, CHUNK_E)],
                              cols_v.at[pl.ds(b0, CHUNK_E)],
                              sem_in.at[slot]).wait()

    def gather_wait(slot):
        pltpu.make_async_copy(x_hbm.at[rfire.at[slot]], vfire.at[slot],
                              sem_g.at[slot]).wait()

    def scatter_wait(slot):
        pltpu.make_async_copy(vfire.at[slot], acc_sh.at[cfire.at[slot]],
                              sem_s.at[slot]).wait()

    def chain_scatter(t):
        @pl.when(t >= LAG)
        def _():
            pslot = lax.rem(t - LAG, FB)
            gather_wait(pslot)
            pltpu.async_copy(vfire.at[pslot], acc_sh.at[cfire.at[pslot]],
                             sem_s.at[pslot], add=True)

    def flush(pcnt, t):
        nfull = pcnt // GW

        def fbody(j, t):
            slot = lax.rem(t, FB)

            @pl.when(t >= FB)
            def _():
                scatter_wait(slot)
            chain_scatter(t)
            for k in range(GW // 16):
                rfire.at[slot][pl.ds(16 * k, 16)] = (
                    rbuf[pl.ds(j * GW + 16 * k, 16)])
                cfire.at[slot][pl.ds(16 * k, 16)] = (
                    cbuf[pl.ds(j * GW + 16 * k, 16)])
            pltpu.async_copy(x_hbm.at[rfire.at[slot]], vfire.at[slot],
                             sem_g.at[slot])
            return t + 1

        t = lax.fori_loop(0, nfull, fbody, t)
        rem = pcnt - nfull * GW

        @pl.when(nfull > 0)
        def _():
            def mv(k, _):
                rbuf[pl.ds(16 * k, 16)] = rbuf[pl.ds(nfull * GW + 16 * k, 16)]
                cbuf[pl.ds(16 * k, 16)] = cbuf[pl.ds(nfull * GW + 16 * k, 16)]
                return 0
            lax.fori_loop(0, (rem + 15) // 16, mv, 0)

        return rem, t

    load_chunk(0, 0)

    def chunk_body(c, carry):
        pcnt, t = carry
        m = lax.rem(c, 3)
        wait_chunk(c, m)

        @pl.when(c + 1 < n_chunks)
        def _():
            load_chunk(c + 1, lax.rem(c + 1, 3))

        b0 = m * CHUNK_E

        @plsc.parallel_loop(0, CHUNK_E // 16, unroll=16, carry=pcnt)
        def pcnt(i, pcnt):
            o = b0 + 16 * i
            row = rows_v[pl.ds(o, 16)]
            col = cols_v[pl.ds(o, 16)]
            fl = plsc.load_gather(flag_v, [col])
            msk = fl > 0
            plsc.store_compressed(rbuf.at[pl.ds(pcnt, 16)], row, mask=msk)
            plsc.store_compressed(cbuf.at[pl.ds(pcnt, 16)], col, mask=msk)
            pcv = plsc.all_reduce_population_count(msk)
            return pcnt + pcv[0]

        pcnt, t = flush(pcnt, t)
        return (pcnt, t)

    pcnt, t = lax.fori_loop(0, n_chunks, chunk_body,
                            (jnp.int32(0), jnp.int32(0)))

    lane = lax.iota(jnp.int32, 16)
    rem16 = lax.rem(pcnt, 16)
    a0 = pcnt - rem16
    keep = lane < rem16
    rbuf[pl.ds(a0, 16)] = jnp.where(keep, rbuf[pl.ds(a0, 16)], 0)
    cbuf[pl.ds(a0, 16)] = jnp.where(keep, cbuf[pl.ds(a0, 16)], DEAD)
    for k in range(1, GW // 16 + 1):
        rbuf[pl.ds(a0 + 16 * k, 16)] = jnp.zeros((16,), jnp.int32)
        cbuf[pl.ds(a0 + 16 * k, 16)] = jnp.full((16,), DEAD, jnp.int32)
    _, t = flush(jnp.where(pcnt > 0, GW, 0), t)

    for d in range(LAG):
        chain_scatter(t + d)
    for s in range(FB):
        @pl.when(s < t)
        def _():
            scatter_wait(jnp.int32(s))
    plsc.subcore_barrier()

    for role_hbm, out_acc, out_xnw in (
        (home_hbm, out_acc_h, out_xnw_h),
        (away_hbm, out_acc_a, out_xnw_a),
    ):
        for q in range(2):
            base = sid * 256 + q * 128
            pltpu.sync_copy(role_hbm.at[pl.ds(base, 128)], idx_v)
            pltpu.sync_copy(acc_sh.at[idx_v], g_v)
            pltpu.sync_copy(g_v, out_acc.at[cid, pl.ds(base, 128)])

            @pl.when(cid == 0)
            def _():
                pltpu.sync_copy(nw_hbm.at[idx_v], nw_v)
                pltpu.sync_copy(x_hbm.at[idx_v], xg_v)
                for k in range(8):
                    xnw_v[pl.ds(16 * k, 16)] = (
                        xg_v[pl.ds(16 * k, 16)] * nw_v[pl.ds(16 * k, 16)])
                pltpu.sync_copy(xnw_v, out_xnw.at[pl.ds(base, 128)])


def kernel(x, node_weight, edge_index, home, away, W, b):
    x_flat = x.reshape(N)
    nw_flat = node_weight.reshape(N)
    zeros_f = jnp.zeros((ACC_PAD,), jnp.float32)
    zeros_i = jnp.zeros((N,), jnp.int32)

    acc_h, acc_a, xnw_h, xnw_a = _sc_pagerank(
        edge_index, x_flat, nw_flat, home, away, zeros_f, zeros_i)

    w00 = W[0, 0]
    th = xnw_h + (acc_h[0] + acc_h[1]) * w00 + b[0]
    ta = xnw_a + (acc_a[0] + acc_a[1]) * w00 + b[0]
    out = jnp.sign(jax.nn.sigmoid(th) - jax.nn.sigmoid(ta))
    return out.reshape(B, 1)
